# Initial kernel scaffold; baseline (speedup 1.0000x reference)
import jax
import jax.numpy as jnp
from jax.experimental import pallas as pl


def kernel(x, edge_index, edge_weight, W, att_src, att_dst, bias, proj_W, proj_b):
    raise NotImplementedError("write your pallas kernel here")


# speedup vs baseline: 32.5525x; 32.5525x over previous
import functools

import jax
import jax.numpy as jnp
from jax import lax
from jax.experimental import pallas as pl
from jax.experimental.pallas import tpu as pltpu
from jax.experimental.pallas import tpu_sc as plsc

N = 50000
E = 800000
H = 2
C = 64
FEAT = H * C
ROW = 144
NEG = 0.2

NC = 2
NS = 16
NW = NC * NS

EPAD = 802816
EPW = EPAD // NS
CHUNK = 1568
NCHUNK = EPW // CHUNK

R = 12288
NPASS = 3
NACC = NPASS * NC * R
RPT = R // NS
WB = 16
NWB = RPT // WB
NPD = NACC + 16

BLK = 1000
GRID = N // BLK


def _pre_body(x_ref, w_ref, asrc_ref, adst_ref, gt_ref, dt_ref):
    x = x_ref[...]
    w = w_ref[...]
    xw = lax.dot_general(x, w, (((1,), (1,)), ((), ())),
                         preferred_element_type=jnp.float32)
    rows = lax.broadcasted_iota(jnp.int32, (FEAT, H), 0) // C
    cols = lax.broadcasted_iota(jnp.int32, (FEAT, H), 1)
    sel = (rows == cols).astype(jnp.float32)
    a_s = lax.dot_general(xw * asrc_ref[...], sel, (((1,), (0,)), ((), ())),
                          preferred_element_type=jnp.float32)
    a_d = lax.dot_general(xw * adst_ref[...], sel, (((1,), (0,)), ((), ())),
                          preferred_element_type=jnp.float32)
    z14 = jnp.zeros((x.shape[0], 14), jnp.float32)
    gt_ref[...] = jnp.concatenate([xw, a_s, z14], axis=1)
    dt_ref[...] = jnp.concatenate([a_d, z14], axis=1)


def _pre(x, attf_src, attf_dst, w):
    return pl.pallas_call(
        _pre_body,
        grid=(GRID,),
        in_specs=[
            pl.BlockSpec((BLK, C), lambda i: (i, 0)),
            pl.BlockSpec((FEAT, C), lambda i: (0, 0)),
            pl.BlockSpec((1, FEAT), lambda i: (0, 0)),
            pl.BlockSpec((1, FEAT), lambda i: (0, 0)),
        ],
        out_specs=[
            pl.BlockSpec((BLK, ROW), lambda i: (i, 0)),
            pl.BlockSpec((BLK, 16), lambda i: (i, 0)),
        ],
        out_shape=[
            jax.ShapeDtypeStruct((N, ROW), jnp.float32),
            jax.ShapeDtypeStruct((N, 16), jnp.float32),
        ],
    )(x, w, attf_src, attf_dst)


def _sc_body(src_hbm, dst_hbm, gt_hbm, dt_hbm, z_hbm, out_hbm,
             accum, schunk, dchunk, csrc, cdst, grow, drow, orows,
             lidx, vbuf, zvbuf, sem0, sem1):
    c = lax.axis_index("c")
    s = lax.axis_index("s")
    ebase = s * EPW
    iota = lax.iota(jnp.int32, 16)
    oh0 = (iota == 0).astype(jnp.float32)
    oh1 = (iota == 1).astype(jnp.float32)

    pltpu.sync_copy(z_hbm, zvbuf)

    def one_pass(p, carry):
        lo = (p * NC + c) * R
        hi = lo + R

        def zero_step(k, carry2):
            pltpu.sync_copy(zvbuf, accum.at[pl.ds(s * RPT + k * WB, WB)])
            return carry2
        lax.fori_loop(0, NWB, zero_step, 0)
        @pl.when(s == 0)
        def _():
            pltpu.sync_copy(zvbuf.at[pl.ds(0, 16)], accum.at[pl.ds(R, 16)])
        plsc.subcore_barrier()

        def chunk_step(ch, carry2):
            pltpu.sync_copy(src_hbm.at[pl.ds(ebase + ch * CHUNK, CHUNK)],
                            schunk)
            pltpu.sync_copy(dst_hbm.at[pl.ds(ebase + ch * CHUNK, CHUNK)],
                            dchunk)

            def vec_step(i, cnt2):
                sv = schunk[pl.ds(i * 16, 16)]
                dv = dchunk[pl.ds(i * 16, 16)]
                m = (dv >= lo) & (dv < hi)
                mi = m.astype(jnp.int32)
                pos = cnt2 - 1 + plsc.cumsum(mi)
                plsc.store_scatter(csrc, [pos], sv, mask=m)
                plsc.store_scatter(cdst, [pos], dv, mask=m)
                return cnt2 + jnp.sum(mi)
            cnt = lax.fori_loop(0, CHUNK // 16, vec_step, jnp.int32(0))

            csrc[pl.ds(cnt, 16)] = jnp.zeros((16,), jnp.int32)
            cdst[pl.ds(cnt, 16)] = jnp.full((16,), lo + R, jnp.int32)
            nb = (cnt + 15) // 16

            def edge_batch(b, carry3):
                pltpu.async_copy(gt_hbm.at[csrc.at[pl.ds(b * 16, 16)]],
                                 grow, sem0).wait()
                pltpu.async_copy(dt_hbm.at[cdst.at[pl.ds(b * 16, 16)]],
                                 drow, sem1).wait()
                a0 = (plsc.load_gather(grow, [iota, jnp.full((16,), FEAT, jnp.int32)])
                      + plsc.load_gather(drow, [iota, jnp.zeros((16,), jnp.int32)]))
                a1 = (plsc.load_gather(grow, [iota, jnp.full((16,), FEAT + 1, jnp.int32)])
                      + plsc.load_gather(drow, [iota, jnp.ones((16,), jnp.int32)]))
                w0 = jnp.exp(jnp.maximum(a0, NEG * a0))
                w1 = jnp.exp(jnp.maximum(a1, NEG * a1))
                for j in range(16):
                    wv0 = jnp.full((16,), w0[j], jnp.float32)
                    wv1 = jnp.full((16,), w1[j], jnp.float32)
                    for cc in range(8):
                        v = grow[j, pl.ds(cc * 16, 16)]
                        orows[j, pl.ds(cc * 16, 16)] = v * (wv0 if cc < 4 else wv1)
                    orows[j, pl.ds(FEAT, 16)] = wv0 * oh0 + wv1 * oh1
                lidx[pl.ds(0, 16)] = cdst[pl.ds(b * 16, 16)] - lo
                pltpu.sync_copy(orows, accum.at[lidx], add=True)
                return carry3
            lax.fori_loop(0, nb, edge_batch, 0)
            return carry2
        lax.fori_loop(0, NCHUNK, chunk_step, 0)
        plsc.subcore_barrier()

        def wb_step(k, carry2):
            r0 = s * RPT + k * WB
            pltpu.sync_copy(accum.at[pl.ds(r0, WB)], vbuf)
            pltpu.sync_copy(vbuf, out_hbm.at[pl.ds(lo + r0, WB)])
            return carry2
        lax.fori_loop(0, NWB, wb_step, 0)
        plsc.subcore_barrier()
        return carry

    lax.fori_loop(0, NPASS, one_pass, 0)


@functools.partial(jax.jit, static_argnums=())
def _sc_edge(src_p, dst_p, gtable, dtable, zeros_wb):
    mesh = plsc.VectorSubcoreMesh(core_axis_name="c", subcore_axis_name="s")
    f = pl.kernel(
        _sc_body,
        mesh=mesh,
        compiler_params=pltpu.CompilerParams(
            use_tc_tiling_on_sc=False, needs_layout_passes=False),
        out_type=jax.ShapeDtypeStruct((NACC, ROW), jnp.float32),
        scratch_types=[
            pltpu.VMEM_SHARED((R + 16, ROW), jnp.float32),
            pltpu.VMEM((CHUNK,), jnp.int32),
            pltpu.VMEM((CHUNK,), jnp.int32),
            pltpu.VMEM((CHUNK + 32,), jnp.int32),
            pltpu.VMEM((CHUNK + 32,), jnp.int32),
            pltpu.VMEM((16, ROW), jnp.float32),
            pltpu.VMEM((16, 16), jnp.float32),
            pltpu.VMEM((16, ROW), jnp.float32),
            pltpu.VMEM((16,), jnp.int32),
            pltpu.VMEM((WB, ROW), jnp.float32),
            pltpu.VMEM((WB, ROW), jnp.float32),
            pltpu.SemaphoreType.DMA,
            pltpu.SemaphoreType.DMA,
        ],
    )
    return f(src_p, dst_p, gtable, dtable, zeros_wb)


def _post_body(sagg_ref, dden_ref, xw_ref, ac_ref, dc_ref,
               bias_ref, pw_ref, pb_ref, out_ref):
    sagg = sagg_ref[...]
    xw = xw_ref[...]
    a0 = ac_ref[:, 0:1] + dc_ref[:, 0:1]
    a1 = ac_ref[:, 1:2] + dc_ref[:, 1:2]
    w0 = jnp.exp(jnp.maximum(a0, NEG * a0))
    w1 = jnp.exp(jnp.maximum(a1, NEG * a1))
    h0 = (sagg[:, 0:C] + w0 * xw[:, 0:C]) / (dden_ref[:, 0:1] + w0 + 1e-16)
    h1 = (sagg[:, C:FEAT] + w1 * xw[:, C:FEAT]) / (dden_ref[:, 1:2] + w1 + 1e-16)
    h = jnp.concatenate([h0, h1], axis=1) + bias_ref[...]
    h = jnp.maximum(h, 0.0)
    y = lax.dot_general(h, pw_ref[...], (((1,), (1,)), ((), ())),
                        preferred_element_type=jnp.float32) + pb_ref[...]
    n2 = jnp.sum(y * y, axis=1, keepdims=True)
    out_ref[...] = y / jnp.maximum(jnp.sqrt(n2), 1e-12)


def _post(sagg, dden, xw, acol, dcol, bias, proj_w, proj_b):
    return pl.pallas_call(
        _post_body,
        grid=(GRID,),
        in_specs=[
            pl.BlockSpec((BLK, FEAT), lambda i: (i, 0)),
            pl.BlockSpec((BLK, 16), lambda i: (i, 0)),
            pl.BlockSpec((BLK, FEAT), lambda i: (i, 0)),
            pl.BlockSpec((BLK, 16), lambda i: (i, 0)),
            pl.BlockSpec((BLK, 16), lambda i: (i, 0)),
            pl.BlockSpec((1, FEAT), lambda i: (0, 0)),
            pl.BlockSpec((C, FEAT), lambda i: (0, 0)),
            pl.BlockSpec((1, C), lambda i: (0, 0)),
        ],
        out_specs=pl.BlockSpec((BLK, C), lambda i: (i, 0)),
        out_shape=jax.ShapeDtypeStruct((N, C), jnp.float32),
    )(sagg, dden, xw, acol, dcol, bias, proj_w, proj_b)


def kernel(x, edge_index, edge_weight, W, att_src, att_dst, bias, proj_W, proj_b):
    del edge_weight
    attf_src = att_src.reshape(1, FEAT)
    attf_dst = att_dst.reshape(1, FEAT)

    gtable, dtab = _pre(x, attf_src, attf_dst, W)
    dtable = jnp.concatenate(
        [dtab, jnp.zeros((NPD - N, 16), jnp.float32)], axis=0)

    src_p = jnp.concatenate(
        [edge_index[0], jnp.zeros((EPAD - E,), jnp.int32)])
    dst_p = jnp.concatenate(
        [edge_index[1], jnp.full((EPAD - E,), jnp.int32(1 << 30))])
    zeros_wb = jnp.zeros((WB, ROW), jnp.float32)

    acc = _sc_edge(src_p, dst_p, gtable, dtable, zeros_wb)

    sagg = acc[:N, :FEAT]
    dden = acc[:N, FEAT:ROW]
    xw = gtable[:, :FEAT]
    acol = gtable[:, FEAT:ROW]

    return _post(sagg, dden, xw, acol, dtab,
                 bias.reshape(1, FEAT), proj_W, proj_b.reshape(1, C))
